# Initial kernel scaffold; baseline (speedup 1.0000x reference)
import jax
import jax.numpy as jnp
from jax.experimental import pallas as pl


def kernel(species, y, W, b, self_energies):
    raise NotImplementedError("write your pallas kernel here")


# speedup vs baseline: 1.6827x; 1.6827x over previous
import functools

import jax
import jax.numpy as jnp
from jax.experimental import pallas as pl

_FEATS = (160, 160, 128, 128)


def _ll_kernel(sp_ref, y_ref, wc_ref, c_ref, out_ref, *, n_atoms, n_elem, kf, inv_nets):
    c_row = c_ref[...]
    acc = jnp.zeros((sp_ref.shape[0], 1), dtype=jnp.float32)
    for a in range(n_atoms):
        ya = y_ref[:, a * kf:(a + 1) * kf]
        oa = jnp.dot(ya, wc_ref[...], preferred_element_type=jnp.float32,
                     precision=jax.lax.Precision.HIGHEST)
        spa = sp_ref[:, a:a + 1]
        eidx = jax.lax.broadcasted_iota(jnp.int32, (spa.shape[0], n_elem), 1)
        onehot = (spa == eidx).astype(jnp.float32)
        acc = acc + jnp.sum((oa * inv_nets + c_row) * onehot, axis=1,
                            keepdims=True)
    out_ref[...] = acc


@jax.jit
def kernel(species, y, W, b, self_energies):
    B, A, NETS, F = y.shape
    N_ELEM = W.shape[1]
    KF = NETS * F

    feats = jnp.asarray(_FEATS[:N_ELEM], dtype=jnp.int32)
    fmask = (jnp.arange(F, dtype=jnp.int32)[None, :] < feats[:, None])
    Wm = W * fmask[None, :, :].astype(W.dtype)
    Wc = Wm.transpose(0, 2, 1).reshape(KF, N_ELEM)
    c = (b.sum(axis=0) / NETS + self_energies)[None, :]

    y3 = y.reshape(B, A * KF)

    BM = 64
    grid = (B // BM,)
    out = pl.pallas_call(
        functools.partial(_ll_kernel, n_atoms=A, n_elem=N_ELEM, kf=KF,
                          inv_nets=1.0 / NETS),
        grid=grid,
        in_specs=[
            pl.BlockSpec((BM, A), lambda m: (m, 0)),
            pl.BlockSpec((BM, A * KF), lambda m: (m, 0)),
            pl.BlockSpec((KF, N_ELEM), lambda m: (0, 0)),
            pl.BlockSpec((1, N_ELEM), lambda m: (0, 0)),
        ],
        out_specs=pl.BlockSpec((BM, 1), lambda m: (m, 0)),
        out_shape=jax.ShapeDtypeStruct((B, 1), jnp.float32),
    )(species, y3, Wc, c)

    return (species, out.reshape(B))
